# baseline (speedup 1.0000x reference)
import functools

import jax
import jax.numpy as jnp
from jax import lax
from jax.experimental import pallas as pl
from jax.experimental.pallas import tpu as pltpu
from jax.experimental.pallas import tpu_sc as plsc

_N, _S, _K, _C = 4096, 20, 10, 200
_NEG = -1000000000.0
_PADID = 204
_CP = 208
_LN2 = 0.6931471805599453
_LOGC = (
    -0.024825606615738415,
    0.2668588228733106,
    -1.234263173084068,
    3.218832837151809,
    -5.264110477180785,
    6.065830143240842,
    -3.0283174810522713,
)

_NW = 32
_NB = _N // _NW


def _ln(sv):
    xi = lax.bitcast_convert_type(sv, jnp.int32)
    ee = ((xi >> 23) - 127).astype(jnp.float32)
    mm = lax.bitcast_convert_type((xi & 0x7FFFFF) | 0x3F800000, jnp.float32)
    p = jnp.full((16,), _LOGC[0], jnp.float32)
    for c in _LOGC[1:]:
        p = p * mm + c
    return (ee + p) * _LN2


def _sc_body(itemsT_hbm, ban_hbm, out_hbm, items_v, ban_v, counts_v,
             e_v, prev_v, outbuf_v):
    wid = lax.axis_index("s") * 2 + lax.axis_index("c")
    base = wid * _NB

    pltpu.sync_copy(itemsT_hbm.at[:, pl.ds(base, _NB)], items_v)
    pltpu.sync_copy(ban_hbm.at[pl.ds(base, _NB)], ban_v)

    iota16 = lax.iota(jnp.int32, 16)
    zeros_f = jnp.zeros((16,), jnp.float32)
    ones_f = jnp.ones((16,), jnp.float32)

    def _zero(n, _):
        for i in range(13):
            counts_v[n, pl.ds(16 * i, 16)] = zeros_f
        return 0

    lax.fori_loop(0, _NB, _zero, 0)
    prev_v[pl.ds(0, 16)] = jnp.full((16,), -1, jnp.int32)

    for g in range(_NB // 16):
        rows = iota16 + 16 * g
        last = jnp.zeros((16,), jnp.int32)
        for j in range(_S):
            idx = items_v[j, pl.ds(16 * g, 16)]
            valid = idx != 0
            plsc.addupdate_scatter(counts_v, [rows, idx], ones_f, mask=valid)
            last = jnp.where(valid, idx, last)
        plsc.addupdate_scatter(counts_v, [rows, last], -ones_f,
                               mask=last != 0)

    def _row(n, _):
        c_regs = [counts_v[n, pl.ds(16 * i, 16)] for i in range(12)]
        c12 = counts_v[n, pl.ds(192, 16)]
        c12b = counts_v[n, pl.ds(184, 16)]

        mx = jnp.maximum(c_regs[0], c12b)
        for i in range(1, 12):
            mx = jnp.maximum(mx, c_regs[i])
        mvec = jnp.full((16,), jnp.max(mx), jnp.float32)

        sacc = zeros_f
        for i in range(12):
            e = jnp.exp(c_regs[i] - mvec)
            e_v[pl.ds(16 * i, 16)] = e
            sacc = sacc + e
        e12 = jnp.where(iota16 >= 8, 0.0, jnp.exp(c12 - mvec))
        e_v[pl.ds(192, 16)] = e12
        sacc = sacc + e12
        sum_all = jnp.sum(sacc)

        for s in range(_S):
            b = ban_v[n, s, pl.ds(0, 16)]
            sk = lax.sort(b)
            plsc.store_scatter(prev_v, [iota16 + 1], sk)
            prev = prev_v[pl.ds(0, 16)]
            eb = plsc.load_gather(e_v, [sk])
            esub = jnp.sum(jnp.where(sk != prev, eb, 0.0))
            sv = jnp.full((16,), sum_all - esub, jnp.float32)
            lse = _ln(sv) + mvec
            for i in range(12):
                outbuf_v[s, pl.ds(16 * i, 16)] = c_regs[i] - lse
            outbuf_v[s, pl.ds(184, 16)] = c12b - lse
            plsc.store_scatter(
                outbuf_v,
                [jnp.full((16,), s, jnp.int32), b],
                jnp.full((16,), _NEG, jnp.float32),
                mask=b < _C,
            )
        pltpu.sync_copy(outbuf_v, out_hbm.at[base + n])
        return 0

    lax.fori_loop(0, _NB, _row, 0)


def kernel(ban_ids, item_ids, aux1, aux2, aux3):
    del aux1, aux2, aux3
    itemsT = item_ids.astype(jnp.int32).T
    ban = ban_ids.astype(jnp.int32)
    ban_pad = jnp.concatenate(
        [ban, jnp.full((_N, _S, 16 - _K), _PADID, jnp.int32)], axis=-1
    )

    mesh = plsc.VectorSubcoreMesh(core_axis_name="c", subcore_axis_name="s")
    run = functools.partial(
        pl.kernel,
        out_type=jax.ShapeDtypeStruct((_N, _S, _C), jnp.float32),
        mesh=mesh,
        compiler_params=pltpu.CompilerParams(
            use_tc_tiling_on_sc=False, needs_layout_passes=False
        ),
        scratch_types=[
            pltpu.VMEM((_S, _NB), jnp.int32),
            pltpu.VMEM((_NB, _S, 16), jnp.int32),
            pltpu.VMEM((_NB, _CP), jnp.float32),
            pltpu.VMEM((_CP,), jnp.float32),
            pltpu.VMEM((24,), jnp.int32),
            pltpu.VMEM((_S, _C), jnp.float32),
        ],
    )(_sc_body)

    pi = run(itemsT, ban_pad)
    v = jnp.zeros((_N, _S, 1), jnp.float32)
    return (pi, v)


# speedup vs baseline: 1.0657x; 1.0657x over previous
import functools

import jax
import jax.numpy as jnp
from jax import lax
from jax.experimental import pallas as pl
from jax.experimental.pallas import tpu as pltpu
from jax.experimental.pallas import tpu_sc as plsc

_N, _S, _K, _C = 4096, 20, 10, 200
_NEG = -1000000000.0
_PADID = 204
_CP = 208
_LN2 = 0.6931471805599453
_LOGC = (
    -0.024825606615738415,
    0.2668588228733106,
    -1.234263173084068,
    3.218832837151809,
    -5.264110477180785,
    6.065830143240842,
    -3.0283174810522713,
)

_NW = 32
_NB = _N // _NW


def _ln(sv):
    xi = lax.bitcast_convert_type(sv, jnp.int32)
    ee = ((xi >> 23) - 127).astype(jnp.float32)
    mm = lax.bitcast_convert_type((xi & 0x7FFFFF) | 0x3F800000, jnp.float32)
    p = jnp.full((16,), _LOGC[0], jnp.float32)
    for c in _LOGC[1:]:
        p = p * mm + c
    return (ee + p) * _LN2


def _sc_body(items_hbm, ban_hbm, out_hbm, items_v, ban_v, counts_v,
             e_v, prev_v, outbuf_v):
    wid = lax.axis_index("s") * 2 + lax.axis_index("c")
    base = wid * _NB

    pltpu.sync_copy(items_hbm.at[pl.ds(base * _S, _NB * _S)], items_v)
    pltpu.sync_copy(ban_hbm.at[pl.ds(base * _S * _K, _NB * _S * _K)],
                    ban_v.at[pl.ds(0, _NB * _S * _K)])

    iota16 = lax.iota(jnp.int32, 16)
    zeros_f = jnp.zeros((16,), jnp.float32)
    ones_f = jnp.ones((16,), jnp.float32)

    def _zero(n, _):
        for i in range(13):
            counts_v[n, pl.ds(16 * i, 16)] = zeros_f
        return 0

    lax.fori_loop(0, _NB, _zero, 0)
    prev_v[pl.ds(0, 16)] = jnp.full((16,), -1, jnp.int32)

    for g in range(_NB // 16):
        rows = iota16 + 16 * g
        rows20 = rows * _S
        last = jnp.zeros((16,), jnp.int32)
        for j in range(_S):
            idx = plsc.load_gather(items_v, [rows20 + j])
            valid = idx != 0
            plsc.addupdate_scatter(counts_v, [rows, idx], ones_f, mask=valid)
            last = jnp.where(valid, idx, last)
        plsc.addupdate_scatter(counts_v, [rows, last], -ones_f,
                               mask=last != 0)

    def _row(n, _):
        c_regs = [counts_v[n, pl.ds(16 * i, 16)] for i in range(12)]
        c12 = counts_v[n, pl.ds(192, 16)]
        c12b = counts_v[n, pl.ds(184, 16)]

        mx = jnp.maximum(c_regs[0], c12b)
        for i in range(1, 12):
            mx = jnp.maximum(mx, c_regs[i])
        mvec = jnp.full((16,), jnp.max(mx), jnp.float32)

        sacc = zeros_f
        for i in range(12):
            e = jnp.exp(c_regs[i] - mvec)
            e_v[pl.ds(16 * i, 16)] = e
            sacc = sacc + e
        e12 = jnp.where(iota16 >= 8, 0.0, jnp.exp(c12 - mvec))
        e_v[pl.ds(192, 16)] = e12
        sacc = sacc + e12
        sum_all = jnp.sum(sacc)

        lane_pad = iota16 >= _K
        bbase = n * (_S * _K) + iota16
        for s in range(_S):
            braw = plsc.load_gather(ban_v, [bbase + s * _K])
            b = jnp.where(lane_pad, _PADID, braw)
            sk = lax.sort(b)
            plsc.store_scatter(prev_v, [iota16 + 1], sk)
            prev = prev_v[pl.ds(0, 16)]
            eb = plsc.load_gather(e_v, [sk])
            esub = jnp.sum(jnp.where(sk != prev, eb, 0.0))
            sv = jnp.full((16,), sum_all - esub, jnp.float32)
            lse = _ln(sv) + mvec
            for i in range(12):
                outbuf_v[s, pl.ds(16 * i, 16)] = c_regs[i] - lse
            outbuf_v[s, pl.ds(184, 16)] = c12b - lse
            plsc.store_scatter(
                outbuf_v,
                [jnp.full((16,), s, jnp.int32), b],
                jnp.full((16,), _NEG, jnp.float32),
                mask=b < _C,
            )
        pltpu.sync_copy(outbuf_v, out_hbm.at[base + n])
        return 0

    lax.fori_loop(0, _NB, _row, 0)


def kernel(ban_ids, item_ids, aux1, aux2, aux3):
    del aux1, aux2, aux3
    items_flat = item_ids.astype(jnp.int32).reshape(-1)
    ban_flat = ban_ids.astype(jnp.int32).reshape(-1)

    mesh = plsc.VectorSubcoreMesh(core_axis_name="c", subcore_axis_name="s")
    run = functools.partial(
        pl.kernel,
        out_type=jax.ShapeDtypeStruct((_N, _S, _C), jnp.float32),
        mesh=mesh,
        compiler_params=pltpu.CompilerParams(
            use_tc_tiling_on_sc=False, needs_layout_passes=False
        ),
        scratch_types=[
            pltpu.VMEM((_NB * _S,), jnp.int32),
            pltpu.VMEM((_NB * _S * _K + 16,), jnp.int32),
            pltpu.VMEM((_NB, _CP), jnp.float32),
            pltpu.VMEM((_CP,), jnp.float32),
            pltpu.VMEM((24,), jnp.int32),
            pltpu.VMEM((_S, _C), jnp.float32),
        ],
    )(_sc_body)

    pi = run(items_flat, ban_flat)
    v = jnp.zeros((_N, _S, 1), jnp.float32)
    return (pi, v)


# speedup vs baseline: 1.1249x; 1.0555x over previous
import functools

import jax
import jax.numpy as jnp
from jax import lax
from jax.experimental import pallas as pl
from jax.experimental.pallas import tpu as pltpu
from jax.experimental.pallas import tpu_sc as plsc

_N, _S, _K, _C = 4096, 20, 10, 200
_NEG = -1000000000.0
_PADID = 204
_CP = 208
_LN2 = 0.6931471805599453
_LOGC = (
    -0.024825606615738415,
    0.2668588228733106,
    -1.234263173084068,
    3.218832837151809,
    -5.264110477180785,
    6.065830143240842,
    -3.0283174810522713,
)

_NW = 32
_NB = _N // _NW


def _ln(sv):
    xi = lax.bitcast_convert_type(sv, jnp.int32)
    ee = ((xi >> 23) - 127).astype(jnp.float32)
    mm = lax.bitcast_convert_type((xi & 0x7FFFFF) | 0x3F800000, jnp.float32)
    p = jnp.full((16,), _LOGC[0], jnp.float32)
    for c in _LOGC[1:]:
        p = p * mm + c
    return (ee + p) * _LN2


def _sc_body(items_hbm, ban_hbm, out_hbm, items_v, ban_v, counts_v,
             e_v, prev_v, outbuf_v, osem):
    wid = lax.axis_index("s") * 2 + lax.axis_index("c")
    base = wid * _NB

    pltpu.sync_copy(items_hbm.at[pl.ds(base * _S, _NB * _S)], items_v)
    pltpu.sync_copy(ban_hbm.at[pl.ds(base * _S * _K, _NB * _S * _K)],
                    ban_v.at[pl.ds(0, _NB * _S * _K)])

    iota16 = lax.iota(jnp.int32, 16)
    zeros_f = jnp.zeros((16,), jnp.float32)
    ones_f = jnp.ones((16,), jnp.float32)

    def _zero(n, _):
        for i in range(13):
            counts_v[n, pl.ds(16 * i, 16)] = zeros_f
        return 0

    lax.fori_loop(0, _NB, _zero, 0)
    for s in range(_S):
        prev_v[pl.ds(24 * s, 16)] = jnp.full((16,), -1, jnp.int32)

    for g in range(_NB // 16):
        rows = iota16 + 16 * g
        rows20 = rows * _S
        last = jnp.zeros((16,), jnp.int32)
        for j in range(_S):
            idx = plsc.load_gather(items_v, [rows20 + j])
            valid = idx != 0
            plsc.addupdate_scatter(counts_v, [rows, idx], ones_f, mask=valid)
            last = jnp.where(valid, idx, last)
        plsc.addupdate_scatter(counts_v, [rows, last], -ones_f,
                               mask=last != 0)

    def _row(n, _):
        c_regs = [counts_v[n, pl.ds(16 * i, 16)] for i in range(12)]
        c12 = counts_v[n, pl.ds(192, 16)]
        c12b = counts_v[n, pl.ds(184, 16)]

        mx = jnp.maximum(c_regs[0], c12b)
        for i in range(1, 12):
            mx = jnp.maximum(mx, c_regs[i])
        mvec = jnp.full((16,), jnp.max(mx), jnp.float32)

        sacc = zeros_f
        for i in range(12):
            e = jnp.exp(c_regs[i] - mvec)
            e_v[pl.ds(16 * i, 16)] = e
            sacc = sacc + e
        e12 = jnp.where(iota16 >= 8, 0.0, jnp.exp(c12 - mvec))
        e_v[pl.ds(192, 16)] = e12
        sacc = sacc + e12
        sum_all = jnp.sum(sacc)

        slot = n & 1

        @pl.when(n >= 2)
        def _wait_prev():
            pltpu.make_async_copy(
                outbuf_v.at[slot], out_hbm.at[base + n], osem.at[slot]
            ).wait()

        lane_pad = iota16 >= _K
        bbase = n * (_S * _K) + iota16
        for s in range(_S):
            braw = plsc.load_gather(ban_v, [bbase + s * _K])
            b = jnp.where(lane_pad, _PADID, braw)
            sk = lax.sort(b)
            plsc.store_scatter(prev_v, [iota16 + 1 + 24 * s], sk)
            prev = prev_v[pl.ds(24 * s, 16)]
            eb = plsc.load_gather(e_v, [sk])
            esub = jnp.sum(jnp.where(sk != prev, eb, 0.0))
            sv = jnp.full((16,), sum_all - esub, jnp.float32)
            lse = _ln(sv) + mvec
            for i in range(12):
                outbuf_v[slot, s, pl.ds(16 * i, 16)] = c_regs[i] - lse
            outbuf_v[slot, s, pl.ds(184, 16)] = c12b - lse
            plsc.store_scatter(
                outbuf_v.at[slot],
                [jnp.full((16,), s, jnp.int32), b],
                jnp.full((16,), _NEG, jnp.float32),
                mask=b < _C,
            )
        pltpu.make_async_copy(
            outbuf_v.at[slot], out_hbm.at[base + n], osem.at[slot]
        ).start()
        return 0

    lax.fori_loop(0, _NB, _row, 0)
    for sl in range(2):
        pltpu.make_async_copy(
            outbuf_v.at[sl], out_hbm.at[base + _NB - 2 + sl], osem.at[sl]
        ).wait()


def kernel(ban_ids, item_ids, aux1, aux2, aux3):
    del aux1, aux2, aux3
    items_flat = item_ids.astype(jnp.int32).reshape(-1)
    ban_flat = ban_ids.astype(jnp.int32).reshape(-1)

    mesh = plsc.VectorSubcoreMesh(core_axis_name="c", subcore_axis_name="s")
    run = functools.partial(
        pl.kernel,
        out_type=jax.ShapeDtypeStruct((_N, _S, _C), jnp.float32),
        mesh=mesh,
        compiler_params=pltpu.CompilerParams(
            use_tc_tiling_on_sc=False, needs_layout_passes=False
        ),
        scratch_types=[
            pltpu.VMEM((_NB * _S,), jnp.int32),
            pltpu.VMEM((_NB * _S * _K + 16,), jnp.int32),
            pltpu.VMEM((_NB, _CP), jnp.float32),
            pltpu.VMEM((_CP,), jnp.float32),
            pltpu.VMEM((24 * _S,), jnp.int32),
            pltpu.VMEM((2, _S, _C), jnp.float32),
            pltpu.SemaphoreType.DMA((2,)),
        ],
    )(_sc_body)

    pi = run(items_flat, ban_flat)
    v = jnp.zeros((_N, _S, 1), jnp.float32)
    return (pi, v)
